# baseline (speedup 1.0000x reference)
import functools

import jax
import jax.numpy as jnp
from jax import lax
from jax.experimental import pallas as pl
from jax.experimental.pallas import tpu as pltpu

B, T, D = 1, 2048, 768
H = 12
HD = D // H
E = 8
HID = int(4 * D * 2 / 3)

TT = 256
TILE = 256
NTILES = 23
NSLOT = NTILES * TILE
NEG = -1e30


def _k1_body(x_ref, g_ref, b_ref, w_ref, o_ref):
    x = x_ref[...]
    mu = jnp.mean(x, axis=1, keepdims=True)
    var = jnp.mean((x - mu) ** 2, axis=1, keepdims=True)
    xn = (x - mu) * lax.rsqrt(var + 1e-5) * g_ref[...] + b_ref[...]
    o_ref[...] = jnp.dot(xn, w_ref[...], preferred_element_type=jnp.float32)


def _qkv(x2, ln1_g, ln1_b, wqkv):
    return pl.pallas_call(
        _k1_body,
        grid=(T // TT,),
        in_specs=[
            pl.BlockSpec((TT, D), lambda i: (i, 0)),
            pl.BlockSpec((1, D), lambda i: (0, 0)),
            pl.BlockSpec((1, D), lambda i: (0, 0)),
            pl.BlockSpec((D, 3 * D), lambda i: (0, 0)),
        ],
        out_specs=pl.BlockSpec((TT, 3 * D), lambda i: (i, 0)),
        out_shape=jax.ShapeDtypeStruct((T, 3 * D), jnp.float32),
    )(x2, ln1_g.reshape(1, D), ln1_b.reshape(1, D), wqkv)


def _k2_body(q_ref, k_ref, v_ref, o_ref):
    i = pl.program_id(1)
    q = q_ref[0]
    k = k_ref[0]
    s = lax.dot_general(q, k, (((1,), (1,)), ((), ())),
                        preferred_element_type=jnp.float32)
    s = s * (1.0 / (HD ** 0.5))
    row = i * TT + lax.broadcasted_iota(jnp.int32, (TT, T), 0)
    col = lax.broadcasted_iota(jnp.int32, (TT, T), 1)
    s = jnp.where(col <= row, s, NEG)
    m = jnp.max(s, axis=1, keepdims=True)
    p = jnp.exp(s - m)
    l = jnp.sum(p, axis=1, keepdims=True)
    o = jnp.dot(p, v_ref[0], preferred_element_type=jnp.float32)
    o_ref[0] = o / l


def _attn(q3, k3, v3):
    return pl.pallas_call(
        _k2_body,
        grid=(H, T // TT),
        in_specs=[
            pl.BlockSpec((1, TT, HD), lambda h, i: (h, i, 0)),
            pl.BlockSpec((1, T, HD), lambda h, i: (h, 0, 0)),
            pl.BlockSpec((1, T, HD), lambda h, i: (h, 0, 0)),
        ],
        out_specs=pl.BlockSpec((1, TT, HD), lambda h, i: (h, i, 0)),
        out_shape=jax.ShapeDtypeStruct((H, T, HD), jnp.float32),
    )(q3, k3, v3)


def _k3_body(ao_ref, x_ref, wp_ref, bp_ref, g_ref, b_ref, wr_ref, br_ref,
             y_ref, xn_ref, meta_ref, cnt_ref, carry):
    i = pl.program_id(0)

    @pl.when(i == 0)
    def _():
        carry[...] = jnp.zeros_like(carry)

    y = jnp.dot(ao_ref[...], wp_ref[...], preferred_element_type=jnp.float32)
    y = y + bp_ref[...] + x_ref[...]
    y_ref[...] = y
    mu = jnp.mean(y, axis=1, keepdims=True)
    var = jnp.mean((y - mu) ** 2, axis=1, keepdims=True)
    xn = (y - mu) * lax.rsqrt(var + 1e-5) * g_ref[...] + b_ref[...]
    xn_ref[...] = xn

    logits = jnp.dot(xn, wr_ref[...], preferred_element_type=jnp.float32)
    logits = logits + br_ref[...]
    lane = lax.broadcasted_iota(jnp.int32, (TT, 128), 1)
    v0 = jnp.max(logits, axis=1, keepdims=True)
    e0 = jnp.min(jnp.where(logits == v0, lane, 128), axis=1, keepdims=True)
    l2 = jnp.where(lane == e0, NEG, logits)
    v1 = jnp.max(l2, axis=1, keepdims=True)
    e1 = jnp.min(jnp.where(l2 == v1, lane, 128), axis=1, keepdims=True)
    bexp = jnp.exp(v1 - v0)
    w0 = 1.0 / (1.0 + bexp)
    w1 = bexp * w0

    m0 = (lane == e0).astype(jnp.float32)
    m1 = (lane == e1).astype(jnp.float32)
    m = m0 + m1
    r = lax.broadcasted_iota(jnp.int32, (TT, TT), 0)
    c = lax.broadcasted_iota(jnp.int32, (TT, TT), 1)
    trilS = (r > c).astype(jnp.float32)
    rank = jnp.dot(trilS, m, preferred_element_type=jnp.float32) + carry[...]
    rank0 = jnp.sum(m0 * rank, axis=1, keepdims=True)
    rank1 = jnp.sum(m1 * rank, axis=1, keepdims=True)
    newc = carry[...] + jnp.sum(m, axis=0, keepdims=True)
    carry[...] = newc
    cnt_ref[...] = newc

    e0f = e0.astype(jnp.float32)
    e1f = e1.astype(jnp.float32)
    meta = (jnp.where(lane == 0, e0f, 0.0) + jnp.where(lane == 1, e1f, 0.0)
            + jnp.where(lane == 2, w0, 0.0) + jnp.where(lane == 3, w1, 0.0)
            + jnp.where(lane == 4, rank0, 0.0) + jnp.where(lane == 5, rank1, 0.0))
    meta_ref[...] = meta


def _proj_router(ao2, x2, wproj, bproj, ln2_g, ln2_b, wr_pad, br_pad):
    return pl.pallas_call(
        _k3_body,
        grid=(T // TT,),
        in_specs=[
            pl.BlockSpec((TT, D), lambda i: (i, 0)),
            pl.BlockSpec((TT, D), lambda i: (i, 0)),
            pl.BlockSpec((D, D), lambda i: (0, 0)),
            pl.BlockSpec((1, D), lambda i: (0, 0)),
            pl.BlockSpec((1, D), lambda i: (0, 0)),
            pl.BlockSpec((1, D), lambda i: (0, 0)),
            pl.BlockSpec((D, 128), lambda i: (0, 0)),
            pl.BlockSpec((1, 128), lambda i: (0, 0)),
        ],
        out_specs=[
            pl.BlockSpec((TT, D), lambda i: (i, 0)),
            pl.BlockSpec((TT, D), lambda i: (i, 0)),
            pl.BlockSpec((TT, 128), lambda i: (i, 0)),
            pl.BlockSpec((1, 128), lambda i: (0, 0)),
        ],
        out_shape=[
            jax.ShapeDtypeStruct((T, D), jnp.float32),
            jax.ShapeDtypeStruct((T, D), jnp.float32),
            jax.ShapeDtypeStruct((T, 128), jnp.float32),
            jax.ShapeDtypeStruct((1, 128), jnp.float32),
        ],
        scratch_shapes=[pltpu.VMEM((1, 128), jnp.float32)],
    )(ao2, x2, wproj, bproj.reshape(1, D), ln2_g.reshape(1, D),
      ln2_b.reshape(1, D), wr_pad, br_pad)


def _k4_body(cnt_ref, meta_ref, meta2_ref, small_ref):
    lane = lax.broadcasted_iota(jnp.int32, (1, 128), 1)
    cnt = cnt_ref[...]
    p = jnp.floor((cnt + (TILE - 1.0)) * (1.0 / TILE)) * TILE
    lt = (lax.broadcasted_iota(jnp.int32, (128, 128), 0)
          < lax.broadcasted_iota(jnp.int32, (128, 128), 1)).astype(jnp.float32)
    P = jnp.dot(p, lt, preferred_element_type=jnp.float32)
    C = P + p
    total = jnp.sum(p, axis=1, keepdims=True)

    lane2 = lax.broadcasted_iota(jnp.int32, (TT, 128), 1)
    meta = meta_ref[...]
    e0f = jnp.sum(jnp.where(lane2 == 0, meta, 0.0), axis=1, keepdims=True)
    e1f = jnp.sum(jnp.where(lane2 == 1, meta, 0.0), axis=1, keepdims=True)
    rank0 = jnp.sum(jnp.where(lane2 == 4, meta, 0.0), axis=1, keepdims=True)
    rank1 = jnp.sum(jnp.where(lane2 == 5, meta, 0.0), axis=1, keepdims=True)
    lane2f = lane2.astype(jnp.float32)
    P0 = jnp.sum(jnp.where(lane2f == e0f, P, 0.0), axis=1, keepdims=True)
    P1 = jnp.sum(jnp.where(lane2f == e1f, P, 0.0), axis=1, keepdims=True)
    d0 = P0 + rank0
    d1 = P1 + rank1
    meta2_ref[...] = (jnp.where(lane2 == 0, d0, 0.0)
                      + jnp.where(lane2 == 1, d1, 0.0))

    thresh = lane.astype(jnp.float32) * TILE
    te = jnp.zeros((1, 128), jnp.float32)
    for e in range(E):
        Ce = jnp.sum(jnp.where(lane == e, C, 0.0), axis=1, keepdims=True)
        te = te + (Ce <= thresh).astype(jnp.float32)
    te = jnp.minimum(te, float(E - 1))
    small_ref[...] = (jnp.where(lane < NTILES, te, 0.0)
                      + jnp.where(lane == 30, total, 0.0))


def _routing_meta(cnt, meta):
    return pl.pallas_call(
        _k4_body,
        grid=(T // TT,),
        in_specs=[
            pl.BlockSpec((1, 128), lambda i: (0, 0)),
            pl.BlockSpec((TT, 128), lambda i: (i, 0)),
        ],
        out_specs=[
            pl.BlockSpec((TT, 128), lambda i: (i, 0)),
            pl.BlockSpec((1, 128), lambda i: (0, 0)),
        ],
        out_shape=[
            jax.ShapeDtypeStruct((T, 128), jnp.float32),
            jax.ShapeDtypeStruct((1, 128), jnp.float32),
        ],
    )(cnt, meta)


def _k6_body(te_ref, gx_ref, w1_ref, w2_ref, w3_ref, b1_ref, b2_ref, b3_ref,
             wb_ref, o_ref):
    gx = gx_ref[...]
    h1 = jnp.dot(gx, w1_ref[0], preferred_element_type=jnp.float32) + b1_ref[0]
    h2 = jnp.dot(gx, w2_ref[0], preferred_element_type=jnp.float32) + b2_ref[0]
    sw = h1 * (h2 * jax.nn.sigmoid(h2))
    o = jnp.dot(sw, w3_ref[0], preferred_element_type=jnp.float32) + b3_ref[0]
    o_ref[...] = o * wb_ref[:, 0:1]


def _experts(te, gx, W1, b1, W2, b2, W3, b3, wexp):
    grid_spec = pltpu.PrefetchScalarGridSpec(
        num_scalar_prefetch=1,
        grid=(NTILES,),
        in_specs=[
            pl.BlockSpec((TILE, D), lambda i, te: (i, 0)),
            pl.BlockSpec((1, D, HID), lambda i, te: (te[i], 0, 0)),
            pl.BlockSpec((1, D, HID), lambda i, te: (te[i], 0, 0)),
            pl.BlockSpec((1, HID, D), lambda i, te: (te[i], 0, 0)),
            pl.BlockSpec((1, 1, HID), lambda i, te: (te[i], 0, 0)),
            pl.BlockSpec((1, 1, HID), lambda i, te: (te[i], 0, 0)),
            pl.BlockSpec((1, 1, D), lambda i, te: (te[i], 0, 0)),
            pl.BlockSpec((TILE, 8), lambda i, te: (i, 0)),
        ],
        out_specs=pl.BlockSpec((TILE, D), lambda i, te: (i, 0)),
    )
    return pl.pallas_call(
        _k6_body,
        grid_spec=grid_spec,
        out_shape=jax.ShapeDtypeStruct((NSLOT, D), jnp.float32),
    )(te, gx, W1, W2, W3, b1.reshape(E, 1, HID), b2.reshape(E, 1, HID),
      b3.reshape(E, 1, D), wexp)


def kernel(x, ln1_g, ln1_b, Wq, Wk, Wv, Wproj, bproj, ln2_g, ln2_b,
           Wr, br, W1, b1, W2, b2, W3, b3):
    x2 = x.reshape(T, D)

    wqkv = jnp.concatenate([
        Wq.transpose(1, 0, 2).reshape(D, D),
        Wk.transpose(1, 0, 2).reshape(D, D),
        Wv.transpose(1, 0, 2).reshape(D, D),
    ], axis=1)
    qkv = _qkv(x2, ln1_g, ln1_b, wqkv)
    q3 = qkv[:, :D].reshape(T, H, HD).transpose(1, 0, 2)
    k3 = qkv[:, D:2 * D].reshape(T, H, HD).transpose(1, 0, 2)
    v3 = qkv[:, 2 * D:].reshape(T, H, HD).transpose(1, 0, 2)

    ao = _attn(q3, k3, v3)
    ao2 = ao.transpose(1, 0, 2).reshape(T, D)

    wr_pad = jnp.zeros((D, 128), jnp.float32).at[:, :E].set(Wr)
    br_pad = jnp.full((1, 128), NEG, jnp.float32).at[0, :E].set(br)
    y, xn2, meta, cnt = _proj_router(ao2, x2, Wproj, bproj, ln2_g, ln2_b,
                                     wr_pad, br_pad)
    meta2, small = _routing_meta(cnt, meta)

    d0 = meta2[:, 0].astype(jnp.int32)
    d1 = meta2[:, 1].astype(jnp.int32)
    w0 = meta[:, 2]
    w1 = meta[:, 3]
    te = small[0, :NTILES].astype(jnp.int32)

    gx = jnp.zeros((NSLOT, D), jnp.float32).at[d0].set(xn2).at[d1].set(xn2)
    wbuf = jnp.zeros((NSLOT,), jnp.float32).at[d0].set(w0).at[d1].set(w1)
    wexp = jnp.broadcast_to(wbuf[:, None], (NSLOT, 8))

    obuf = _experts(te, gx, W1, b1, W2, b2, W3, b3, wexp)

    out = y + obuf[d0] + obuf[d1]
    return out.reshape(B, T, D)
